# baseline (speedup 1.0000x reference)
import functools

import jax
import jax.numpy as jnp
from jax import lax
from jax.experimental import pallas as pl
from jax.experimental.pallas import tpu as pltpu
from jax.experimental.pallas import tpu_sc as plsc

DIM = 64
CHUNK = 128
NBUF = 4


@functools.cache
def _build(n_b: int, n_t: int, vocab: int):
    info = plsc.get_sparse_core_info()
    nc = info.num_cores
    nw = nc * info.num_subcores
    b_per_w = n_b // nw
    cpt = b_per_w // CHUNK
    n_chunks = n_t * cpt
    n_outer = n_chunks // NBUF
    assert b_per_w % CHUNK == 0 and n_chunks % NBUF == 0

    mesh = plsc.VectorSubcoreMesh(core_axis_name="c", subcore_axis_name="s")

    @functools.partial(
        pl.kernel,
        mesh=mesh,
        out_type=jax.ShapeDtypeStruct((n_t, DIM // 8, n_b // 128, 8, 128), jnp.float32),
        scratch_types=[
            pltpu.VMEM((n_t, b_per_w), jnp.int32),
            pltpu.VMEM((NBUF, CHUNK, DIM), jnp.float32),
            pltpu.VMEM((DIM // 8, 8, 128), jnp.float32),
            [pltpu.SemaphoreType.DMA] * NBUF,
        ],
        compiler_params=pltpu.CompilerParams(
            use_tc_tiling_on_sc=False, needs_layout_passes=False
        ),
    )
    def gather_kernel(ids_hbm, table_hbm, out_hbm, idx_v, rows_v, tbuf, gsems):
        wid = lax.axis_index("s") * nc + lax.axis_index("c")
        b0 = wid * b_per_w
        pltpu.sync_copy(ids_hbm.at[:, pl.ds(b0, b_per_w)], idx_v)

        def fire(g, b):
            t = g // cpt
            j = g - t * cpt
            pltpu.async_copy(
                table_hbm.at[idx_v.at[t].at[pl.ds(j * CHUNK, CHUNK)]],
                rows_v.at[b],
                gsems[b],
            )

        lane = lax.iota(jnp.int32, 16)

        def drain_write(g, b):
            t = g // cpt
            j = g - t * cpt
            pltpu.make_async_copy(
                table_hbm.at[idx_v.at[t].at[pl.ds(j * CHUNK, CHUNK)]],
                rows_v.at[b],
                gsems[b],
            ).wait()

            def tpose(c8, carry):
                base = jnp.zeros((16,), jnp.int32) + c8 * 8
                for ci in range(8):
                    col = base + ci
                    for bg in range(CHUNK // 16):
                        v = plsc.load_gather(rows_v.at[b], [bg * 16 + lane, col])
                        tbuf.at[c8].at[ci][pl.ds(bg * 16, 16)] = v
                return carry

            lax.fori_loop(0, DIM // 8, tpose, 0)
            bb = (b0 + j * CHUNK) // 128
            pltpu.sync_copy(tbuf, out_hbm.at[t, :, bb])

        for b in range(NBUF):
            fire(b, b)

        def outer(i, carry):
            for b in range(NBUF):
                g = i * NBUF + b
                drain_write(g, b)
                fire(g + NBUF, b)
            return carry

        lax.fori_loop(0, n_outer - 1, outer, 0)
        for b in range(NBUF):
            drain_write((n_outer - 1) * NBUF + b, b)

    return gather_kernel


def kernel(ids, emb_matrix):
    n_b, n_t = ids.shape
    vocab, dim = emb_matrix.shape
    ids_t = jnp.swapaxes(ids, 0, 1).astype(jnp.int32)
    out5 = _build(n_b, n_t, vocab)(ids_t, emb_matrix)
    return out5.transpose(2, 4, 0, 1, 3).reshape(n_b, n_t, dim)


# speedup vs baseline: 1.3638x; 1.3638x over previous
import functools

import jax
import jax.numpy as jnp
from jax import lax
from jax.experimental import pallas as pl
from jax.experimental.pallas import tpu as pltpu
from jax.experimental.pallas import tpu_sc as plsc

DIM = 64
CHUNK = 128
NBUF = 4


@functools.cache
def _build(n_b: int, n_t: int, vocab: int):
    info = plsc.get_sparse_core_info()
    nc = info.num_cores
    nw = nc * info.num_subcores
    b_per_w = n_b // nw
    cpt = b_per_w // CHUNK
    n_chunks = n_t * cpt
    n_outer = n_chunks // NBUF
    assert b_per_w % CHUNK == 0 and n_chunks % NBUF == 0

    mesh = plsc.VectorSubcoreMesh(core_axis_name="c", subcore_axis_name="s")

    @functools.partial(
        pl.kernel,
        mesh=mesh,
        out_type=jax.ShapeDtypeStruct((n_t, n_b, DIM), jnp.float32),
        scratch_types=[
            pltpu.VMEM((n_t, b_per_w), jnp.int32),
            pltpu.VMEM((NBUF, CHUNK, DIM), jnp.float32),
            [pltpu.SemaphoreType.DMA] * NBUF,
        ],
        compiler_params=pltpu.CompilerParams(use_tc_tiling_on_sc=False),
    )
    def gather_kernel(ids_hbm, table_hbm, out_hbm, idx_v, rows_v, gsems):
        wid = lax.axis_index("s") * nc + lax.axis_index("c")
        b0 = wid * b_per_w
        pltpu.sync_copy(ids_hbm.at[:, pl.ds(b0, b_per_w)], idx_v)

        def fire(g, b):
            t = g // cpt
            j = g - t * cpt
            pltpu.async_copy(
                table_hbm.at[idx_v.at[t].at[pl.ds(j * CHUNK, CHUNK)]],
                rows_v.at[b],
                gsems[b],
            )

        def drain_write(g, b):
            t = g // cpt
            j = g - t * cpt
            pltpu.make_async_copy(
                table_hbm.at[idx_v.at[t].at[pl.ds(j * CHUNK, CHUNK)]],
                rows_v.at[b],
                gsems[b],
            ).wait()
            pltpu.sync_copy(
                rows_v.at[b],
                out_hbm.at[t].at[pl.ds(b0 + j * CHUNK, CHUNK)],
            )

        for b in range(NBUF):
            fire(b, b)

        def outer(i, carry):
            for b in range(NBUF):
                g = i * NBUF + b
                drain_write(g, b)
                fire(g + NBUF, b)
            return carry

        lax.fori_loop(0, n_outer - 1, outer, 0)
        for b in range(NBUF):
            drain_write((n_outer - 1) * NBUF + b, b)

    return gather_kernel


def kernel(ids, emb_matrix):
    n_b, n_t = ids.shape
    vocab, dim = emb_matrix.shape
    ids_t = jnp.swapaxes(ids, 0, 1).astype(jnp.int32)
    out3 = _build(n_b, n_t, vocab)(ids_t, emb_matrix)
    return jnp.swapaxes(out3, 0, 1)


# speedup vs baseline: 1.3890x; 1.0185x over previous
import functools

import jax
import jax.numpy as jnp
from jax import lax
from jax.experimental import pallas as pl
from jax.experimental.pallas import tpu as pltpu
from jax.experimental.pallas import tpu_sc as plsc

DIM = 64
CHUNK = 128
NBUF = 4


@functools.cache
def _build(n_b: int, n_t: int, vocab: int):
    info = plsc.get_sparse_core_info()
    nc = info.num_cores
    nw = nc * info.num_subcores
    b_per_w = n_b // nw
    cpt = b_per_w // CHUNK
    n_chunks = n_t * cpt
    n_outer = n_chunks // NBUF
    assert b_per_w % CHUNK == 0 and n_chunks % NBUF == 0

    mesh = plsc.VectorSubcoreMesh(core_axis_name="c", subcore_axis_name="s")

    @functools.partial(
        pl.kernel,
        mesh=mesh,
        out_type=jax.ShapeDtypeStruct((n_t, n_b, DIM), jnp.float32),
        scratch_types=[
            pltpu.VMEM((n_t, b_per_w), jnp.int32),
            pltpu.VMEM((NBUF, CHUNK, DIM), jnp.float32),
            [pltpu.SemaphoreType.DMA] * NBUF,
        ],
        compiler_params=pltpu.CompilerParams(use_tc_tiling_on_sc=False),
    )
    def gather_kernel(ids_hbm, table_hbm, out_hbm, idx_v, rows_v, gsems):
        wid = lax.axis_index("s") * nc + lax.axis_index("c")
        b0 = wid * b_per_w
        pltpu.sync_copy(ids_hbm.at[:, pl.ds(b0, b_per_w)], idx_v)

        def fire(g, b):
            t = g // cpt
            j = g - t * cpt
            pltpu.async_copy(
                table_hbm.at[idx_v.at[t].at[pl.ds(j * CHUNK, CHUNK)]],
                rows_v.at[b],
                gsems[b],
            )

        def drain_write(g, b):
            t = g // cpt
            j = g - t * cpt
            pltpu.make_async_copy(
                table_hbm.at[idx_v.at[t].at[pl.ds(j * CHUNK, CHUNK)]],
                rows_v.at[b],
                gsems[b],
            ).wait()
            pltpu.sync_copy(
                rows_v.at[b],
                out_hbm.at[t].at[pl.ds(b0 + j * CHUNK, CHUNK)],
            )

        for b in range(NBUF):
            fire(b, b)

        def outer(i, carry):
            for b in range(NBUF):
                g = i * NBUF + b
                drain_write(g, b)
                fire(g + NBUF, b)
            return carry

        lax.fori_loop(0, n_outer - 1, outer, 0)
        for b in range(NBUF):
            drain_write((n_outer - 1) * NBUF + b, b)

    return gather_kernel


def _linearize_table(emb):
    v, d = emb.shape
    br = 8000

    def body(x_ref, o_ref):
        even = x_ref[pl.Slice(0, br // 2, 2), :]
        odd = x_ref[pl.Slice(1, br // 2, 2), :]
        o_ref[...] = jnp.concatenate([even, odd], axis=1)

    return pl.pallas_call(
        body,
        grid=(v // br,),
        in_specs=[pl.BlockSpec((br, d), lambda i: (i, 0))],
        out_specs=pl.BlockSpec((br // 2, 2 * d), lambda i: (i, 0)),
        out_shape=jax.ShapeDtypeStruct((v // 2, 2 * d), jnp.float32),
    )(emb)


def kernel(ids, emb_matrix):
    n_b, n_t = ids.shape
    vocab, dim = emb_matrix.shape
    ids_t = jnp.swapaxes(ids, 0, 1).astype(jnp.int32)
    table_lin = _linearize_table(emb_matrix).reshape(vocab, dim)
    out3 = _build(n_b, n_t, vocab)(ids_t, table_lin)
    return jnp.swapaxes(out3, 0, 1)


# speedup vs baseline: 1.6561x; 1.1923x over previous
import functools

import jax
import jax.numpy as jnp
from jax import lax
from jax.experimental import pallas as pl
from jax.experimental.pallas import tpu as pltpu
from jax.experimental.pallas import tpu_sc as plsc

DIM = 64
CHUNK = 128
NBUF = 4


@functools.cache
def _build(n_b: int, n_t: int, vocab: int):
    info = plsc.get_sparse_core_info()
    nc = info.num_cores
    nw = nc * info.num_subcores
    b_per_w = n_b // nw
    cpt = b_per_w // CHUNK
    n_chunks = n_t * cpt
    n_outer = n_chunks // NBUF
    assert b_per_w % CHUNK == 0 and n_chunks % NBUF == 0

    mesh = plsc.VectorSubcoreMesh(core_axis_name="c", subcore_axis_name="s")

    @functools.partial(
        pl.kernel,
        mesh=mesh,
        out_type=jax.ShapeDtypeStruct((n_t, n_b, DIM), jnp.float32),
        scratch_types=[
            pltpu.VMEM((n_t, b_per_w), jnp.int32),
            pltpu.VMEM((NBUF, CHUNK, DIM), jnp.float32),
            [pltpu.SemaphoreType.DMA] * NBUF,
        ],
        compiler_params=pltpu.CompilerParams(use_tc_tiling_on_sc=False),
    )
    def gather_kernel(ids_hbm, table_hbm, out_hbm, idx_v, rows_v, gsems):
        wid = lax.axis_index("s") * nc + lax.axis_index("c")
        b0 = wid * b_per_w
        pltpu.sync_copy(ids_hbm.at[:, pl.ds(b0, b_per_w)], idx_v)

        def fire(g, b):
            t = g // cpt
            j = g - t * cpt
            pltpu.async_copy(
                table_hbm.at[idx_v.at[t].at[pl.ds(j * CHUNK, CHUNK)]],
                rows_v.at[b],
                gsems[b],
            )

        def drain_write(g, b):
            t = g // cpt
            j = g - t * cpt
            pltpu.make_async_copy(
                table_hbm.at[idx_v.at[t].at[pl.ds(j * CHUNK, CHUNK)]],
                rows_v.at[b],
                gsems[b],
            ).wait()
            pltpu.sync_copy(
                rows_v.at[b],
                out_hbm.at[t].at[pl.ds(b0 + j * CHUNK, CHUNK)],
            )

        for b in range(NBUF):
            fire(b, b)

        def outer(i, carry):
            for b in range(NBUF):
                g = i * NBUF + b
                drain_write(g, b)
                fire(g + NBUF, b)
            return carry

        lax.fori_loop(0, n_outer - 1, outer, 0)
        for b in range(NBUF):
            drain_write((n_outer - 1) * NBUF + b, b)

    return gather_kernel


def _linearize_table(emb):
    v, d = emb.shape
    embt = jnp.swapaxes(emb, 0, 1)
    bc = 2048

    def body(x_ref, o_ref, tmp_ref):
        tmp_ref[...] = x_ref[...].T
        even = tmp_ref[pl.Slice(0, bc // 2, 2), :]
        odd = tmp_ref[pl.Slice(1, bc // 2, 2), :]
        o_ref[...] = jnp.concatenate([even, odd], axis=1)

    return pl.pallas_call(
        body,
        grid=((v + bc - 1) // bc,),
        in_specs=[pl.BlockSpec((d, bc), lambda i: (0, i))],
        out_specs=pl.BlockSpec((bc // 2, 2 * d), lambda i: (i, 0)),
        out_shape=jax.ShapeDtypeStruct((v // 2, 2 * d), jnp.float32),
        scratch_shapes=[pltpu.VMEM((bc, d), jnp.float32)],
    )(embt)


def kernel(ids, emb_matrix):
    n_b, n_t = ids.shape
    vocab, dim = emb_matrix.shape
    ids_t = jnp.swapaxes(ids, 0, 1).astype(jnp.int32)
    table_lin = _linearize_table(emb_matrix).reshape(vocab, dim)
    out3 = _build(n_b, n_t, vocab)(ids_t, table_lin)
    return jnp.swapaxes(out3, 0, 1)


# speedup vs baseline: 1.9955x; 1.2049x over previous
import functools

import jax
import jax.numpy as jnp
from jax import lax
from jax.experimental import pallas as pl
from jax.experimental.pallas import tpu as pltpu
from jax.experimental.pallas import tpu_sc as plsc

DIM = 64
CHUNK = 128
NBUF = 4


@functools.cache
def _build(n_b: int, n_t: int, vocab: int):
    info = plsc.get_sparse_core_info()
    nc = info.num_cores
    nw = nc * info.num_subcores
    b_per_w = n_b // nw
    cpt = b_per_w // CHUNK
    n_chunks = n_t * cpt
    n_outer = n_chunks // NBUF
    assert b_per_w % CHUNK == 0 and n_chunks % NBUF == 0

    mesh = plsc.VectorSubcoreMesh(core_axis_name="c", subcore_axis_name="s")

    @functools.partial(
        pl.kernel,
        mesh=mesh,
        out_type=jax.ShapeDtypeStruct((n_t, n_b, DIM), jnp.float32),
        scratch_types=[
            pltpu.VMEM((n_t, b_per_w), jnp.int32),
            pltpu.VMEM((NBUF, CHUNK, DIM), jnp.float32),
            [pltpu.SemaphoreType.DMA] * NBUF,
        ],
        compiler_params=pltpu.CompilerParams(use_tc_tiling_on_sc=False),
    )
    def gather_kernel(ids_hbm, table_hbm, out_hbm, idx_v, rows_v, gsems):
        wid = lax.axis_index("s") * nc + lax.axis_index("c")
        b0 = wid * b_per_w
        pltpu.sync_copy(ids_hbm.at[:, pl.ds(b0, b_per_w)], idx_v)

        def fire(g, b):
            t = g // cpt
            j = g - t * cpt
            pltpu.async_copy(
                table_hbm.at[idx_v.at[t].at[pl.ds(j * CHUNK, CHUNK)]],
                rows_v.at[b],
                gsems[b],
            )

        def drain_write(g, b):
            t = g // cpt
            j = g - t * cpt
            pltpu.make_async_copy(
                table_hbm.at[idx_v.at[t].at[pl.ds(j * CHUNK, CHUNK)]],
                rows_v.at[b],
                gsems[b],
            ).wait()
            pltpu.sync_copy(
                rows_v.at[b],
                out_hbm.at[t].at[pl.ds(b0 + j * CHUNK, CHUNK)],
            )

        for b in range(NBUF):
            fire(b, b)

        def outer(i, carry):
            for b in range(NBUF):
                g = i * NBUF + b
                drain_write(g, b)
                fire(g + NBUF, b)
            return carry

        lax.fori_loop(0, n_outer - 1, outer, 0)
        for b in range(NBUF):
            drain_write((n_outer - 1) * NBUF + b, b)

    return gather_kernel


def _linearize_table(emb):
    v, d = emb.shape
    embt = jnp.swapaxes(emb, 0, 1)
    bc = 4096

    def body(x_ref, o_ref, tmp_ref):
        tmp_ref[...] = x_ref[...].T
        even = tmp_ref[pl.Slice(0, bc // 2, 2), :]
        odd = tmp_ref[pl.Slice(1, bc // 2, 2), :]
        o_ref[...] = jnp.concatenate([even, odd], axis=1)

    return pl.pallas_call(
        body,
        grid=((v + bc - 1) // bc,),
        in_specs=[pl.BlockSpec((d, bc), lambda i: (0, i))],
        out_specs=pl.BlockSpec((bc // 2, 2 * d), lambda i: (i, 0)),
        out_shape=jax.ShapeDtypeStruct((v // 2, 2 * d), jnp.float32),
        scratch_shapes=[pltpu.VMEM((bc, d), jnp.float32)],
    )(embt)


def kernel(ids, emb_matrix):
    n_b, n_t = ids.shape
    vocab, dim = emb_matrix.shape
    ids_t = jnp.swapaxes(ids, 0, 1).astype(jnp.int32)
    table_lin = _linearize_table(emb_matrix).reshape(vocab, dim)
    out3 = _build(n_b, n_t, vocab)(ids_t, table_lin)
    return jnp.swapaxes(out3, 0, 1)


# speedup vs baseline: 2.2567x; 1.1309x over previous
import functools

import jax
import jax.numpy as jnp
from jax import lax
from jax.experimental import pallas as pl
from jax.experimental.pallas import tpu as pltpu
from jax.experimental.pallas import tpu_sc as plsc

DIM = 64
CHUNK = 128
NBUF = 4


@functools.cache
def _build(n_b: int, n_t: int, vocab: int):
    info = plsc.get_sparse_core_info()
    nc = info.num_cores
    nw = nc * info.num_subcores
    b_per_w = n_b // nw
    cpt = b_per_w // CHUNK
    n_chunks = n_t * cpt
    n_outer = n_chunks // NBUF
    assert b_per_w % CHUNK == 0 and n_chunks % NBUF == 0

    mesh = plsc.VectorSubcoreMesh(core_axis_name="c", subcore_axis_name="s")

    @functools.partial(
        pl.kernel,
        mesh=mesh,
        out_type=jax.ShapeDtypeStruct((n_t, n_b, DIM), jnp.float32),
        scratch_types=[
            pltpu.VMEM((n_t, b_per_w), jnp.int32),
            pltpu.VMEM((NBUF, CHUNK, DIM), jnp.float32),
            [pltpu.SemaphoreType.DMA] * NBUF,
        ],
        compiler_params=pltpu.CompilerParams(use_tc_tiling_on_sc=False),
    )
    def gather_kernel(ids_hbm, table_hbm, out_hbm, idx_v, rows_v, gsems):
        wid = lax.axis_index("s") * nc + lax.axis_index("c")
        b0 = wid * b_per_w
        pltpu.sync_copy(ids_hbm.at[:, pl.ds(b0, b_per_w)], idx_v)

        def fire(g, b):
            t = g // cpt
            j = g - t * cpt
            pltpu.async_copy(
                table_hbm.at[idx_v.at[t].at[pl.ds(j * CHUNK, CHUNK)]],
                rows_v.at[b],
                gsems[b],
            )

        def drain_write(g, b):
            t = g // cpt
            j = g - t * cpt
            pltpu.make_async_copy(
                table_hbm.at[idx_v.at[t].at[pl.ds(j * CHUNK, CHUNK)]],
                rows_v.at[b],
                gsems[b],
            ).wait()
            pltpu.sync_copy(
                rows_v.at[b],
                out_hbm.at[t].at[pl.ds(b0 + j * CHUNK, CHUNK)],
            )

        for b in range(NBUF):
            fire(b, b)

        def outer(i, carry):
            for b in range(NBUF):
                g = i * NBUF + b
                drain_write(g, b)
                fire(g + NBUF, b)
            return carry

        lax.fori_loop(0, n_outer - 1, outer, 0)
        for b in range(NBUF):
            drain_write((n_outer - 1) * NBUF + b, b)

    return gather_kernel


def _linearize_table(emb):
    v, d = emb.shape
    embt = jnp.swapaxes(emb, 0, 1)
    bc = 8192

    def body(x_ref, o_ref, tmp_ref):
        tmp_ref[...] = x_ref[...].T
        even = tmp_ref[pl.Slice(0, bc // 2, 2), :]
        odd = tmp_ref[pl.Slice(1, bc // 2, 2), :]
        o_ref[...] = jnp.concatenate([even, odd], axis=1)

    return pl.pallas_call(
        body,
        grid=((v + bc - 1) // bc,),
        in_specs=[pl.BlockSpec((d, bc), lambda i: (0, i))],
        out_specs=pl.BlockSpec((bc // 2, 2 * d), lambda i: (i, 0)),
        out_shape=jax.ShapeDtypeStruct((v // 2, 2 * d), jnp.float32),
        scratch_shapes=[pltpu.VMEM((bc, d), jnp.float32)],
    )(embt)


def kernel(ids, emb_matrix):
    n_b, n_t = ids.shape
    vocab, dim = emb_matrix.shape
    ids_t = jnp.swapaxes(ids, 0, 1).astype(jnp.int32)
    table_lin = _linearize_table(emb_matrix).reshape(vocab, dim)
    out3 = _build(n_b, n_t, vocab)(ids_t, table_lin)
    return jnp.swapaxes(out3, 0, 1)


# speedup vs baseline: 2.4111x; 1.0684x over previous
import functools

import jax
import jax.numpy as jnp
from jax import lax
from jax.experimental import pallas as pl
from jax.experimental.pallas import tpu as pltpu
from jax.experimental.pallas import tpu_sc as plsc

DIM = 64
CHUNK = 128
NBUF = 4


@functools.cache
def _build(n_b: int, n_t: int, vocab: int):
    info = plsc.get_sparse_core_info()
    nc = info.num_cores
    nw = nc * info.num_subcores
    b_per_w = n_b // nw
    cpt = b_per_w // CHUNK
    n_chunks = n_t * cpt
    n_outer = n_chunks // NBUF
    assert b_per_w % CHUNK == 0 and n_chunks % NBUF == 0

    mesh = plsc.VectorSubcoreMesh(core_axis_name="c", subcore_axis_name="s")

    @functools.partial(
        pl.kernel,
        mesh=mesh,
        out_type=jax.ShapeDtypeStruct((n_t, n_b, DIM), jnp.float32),
        scratch_types=[
            pltpu.VMEM((n_t, b_per_w), jnp.int32),
            pltpu.VMEM((NBUF, CHUNK, DIM), jnp.float32),
            [pltpu.SemaphoreType.DMA] * NBUF,
        ],
        compiler_params=pltpu.CompilerParams(use_tc_tiling_on_sc=False),
    )
    def gather_kernel(ids_hbm, table_hbm, out_hbm, idx_v, rows_v, gsems):
        wid = lax.axis_index("s") * nc + lax.axis_index("c")
        b0 = wid * b_per_w
        pltpu.sync_copy(ids_hbm.at[:, pl.ds(b0, b_per_w)], idx_v)

        def fire(g, b):
            t = g // cpt
            j = g - t * cpt
            pltpu.async_copy(
                table_hbm.at[idx_v.at[t].at[pl.ds(j * CHUNK, CHUNK)]],
                rows_v.at[b],
                gsems[b],
            )

        def drain_write(g, b):
            t = g // cpt
            j = g - t * cpt
            pltpu.make_async_copy(
                table_hbm.at[idx_v.at[t].at[pl.ds(j * CHUNK, CHUNK)]],
                rows_v.at[b],
                gsems[b],
            ).wait()
            pltpu.sync_copy(
                rows_v.at[b],
                out_hbm.at[t].at[pl.ds(b0 + j * CHUNK, CHUNK)],
            )

        for b in range(NBUF):
            fire(b, b)

        def outer(i, carry):
            for b in range(NBUF):
                g = i * NBUF + b
                drain_write(g, b)
                fire(g + NBUF, b)
            return carry

        lax.fori_loop(0, n_outer - 1, outer, 0)
        for b in range(NBUF):
            drain_write((n_outer - 1) * NBUF + b, b)

    return gather_kernel


def _linearize_table(emb):
    v, d = emb.shape
    embt = jnp.swapaxes(emb, 0, 1)
    bc = 16384

    def body(x_ref, o_ref, tmp_ref):
        tmp_ref[...] = x_ref[...].T
        even = tmp_ref[pl.Slice(0, bc // 2, 2), :]
        odd = tmp_ref[pl.Slice(1, bc // 2, 2), :]
        o_ref[...] = jnp.concatenate([even, odd], axis=1)

    return pl.pallas_call(
        body,
        grid=((v + bc - 1) // bc,),
        in_specs=[pl.BlockSpec((d, bc), lambda i: (0, i))],
        out_specs=pl.BlockSpec((bc // 2, 2 * d), lambda i: (i, 0)),
        out_shape=jax.ShapeDtypeStruct((v // 2, 2 * d), jnp.float32),
        scratch_shapes=[pltpu.VMEM((bc, d), jnp.float32)],
    )(embt)


def kernel(ids, emb_matrix):
    n_b, n_t = ids.shape
    vocab, dim = emb_matrix.shape
    ids_t = jnp.swapaxes(ids, 0, 1).astype(jnp.int32)
    table_lin = _linearize_table(emb_matrix).reshape(vocab, dim)
    out3 = _build(n_b, n_t, vocab)(ids_t, table_lin)
    return jnp.swapaxes(out3, 0, 1)


# speedup vs baseline: 2.4882x; 1.0320x over previous
import functools

import jax
import jax.numpy as jnp
from jax import lax
from jax.experimental import pallas as pl
from jax.experimental.pallas import tpu as pltpu
from jax.experimental.pallas import tpu_sc as plsc

DIM = 64
CHUNK = 128
NBUF = 4


@functools.cache
def _build(n_b: int, n_t: int, vocab: int):
    info = plsc.get_sparse_core_info()
    nc = info.num_cores
    nw = nc * info.num_subcores
    b_per_w = n_b // nw
    cpt = b_per_w // CHUNK
    n_chunks = n_t * cpt
    n_outer = n_chunks // NBUF
    assert b_per_w % CHUNK == 0 and n_chunks % NBUF == 0

    mesh = plsc.VectorSubcoreMesh(core_axis_name="c", subcore_axis_name="s")

    @functools.partial(
        pl.kernel,
        mesh=mesh,
        out_type=jax.ShapeDtypeStruct((n_t, n_b, DIM), jnp.float32),
        scratch_types=[
            pltpu.VMEM((n_t, b_per_w), jnp.int32),
            pltpu.VMEM((NBUF, CHUNK, DIM), jnp.float32),
            [pltpu.SemaphoreType.DMA] * NBUF,
        ],
        compiler_params=pltpu.CompilerParams(use_tc_tiling_on_sc=False),
    )
    def gather_kernel(ids_hbm, table_hbm, out_hbm, idx_v, rows_v, gsems):
        wid = lax.axis_index("s") * nc + lax.axis_index("c")
        b0 = wid * b_per_w
        pltpu.sync_copy(ids_hbm.at[:, pl.ds(b0, b_per_w)], idx_v)

        def fire(g, b):
            t = g // cpt
            j = g - t * cpt
            pltpu.async_copy(
                table_hbm.at[idx_v.at[t].at[pl.ds(j * CHUNK, CHUNK)]],
                rows_v.at[b],
                gsems[b],
            )

        def drain_write(g, b):
            t = g // cpt
            j = g - t * cpt
            pltpu.make_async_copy(
                table_hbm.at[idx_v.at[t].at[pl.ds(j * CHUNK, CHUNK)]],
                rows_v.at[b],
                gsems[b],
            ).wait()
            pltpu.sync_copy(
                rows_v.at[b],
                out_hbm.at[t].at[pl.ds(b0 + j * CHUNK, CHUNK)],
            )

        for b in range(NBUF):
            fire(b, b)

        def outer(i, carry):
            for b in range(NBUF):
                g = i * NBUF + b
                drain_write(g, b)
                fire(g + NBUF, b)
            return carry

        lax.fori_loop(0, n_outer - 1, outer, 0)
        for b in range(NBUF):
            drain_write((n_outer - 1) * NBUF + b, b)

    return gather_kernel


def _linearize_table(emb):
    v, d = emb.shape
    embt = jnp.swapaxes(emb, 0, 1)
    bc = 32768

    def body(x_ref, o_ref, tmp_ref):
        tmp_ref[...] = x_ref[...].T
        even = tmp_ref[pl.Slice(0, bc // 2, 2), :]
        odd = tmp_ref[pl.Slice(1, bc // 2, 2), :]
        o_ref[...] = jnp.concatenate([even, odd], axis=1)

    return pl.pallas_call(
        body,
        grid=((v + bc - 1) // bc,),
        in_specs=[pl.BlockSpec((d, bc), lambda i: (0, i))],
        out_specs=pl.BlockSpec((bc // 2, 2 * d), lambda i: (i, 0)),
        out_shape=jax.ShapeDtypeStruct((v // 2, 2 * d), jnp.float32),
        scratch_shapes=[pltpu.VMEM((bc, d), jnp.float32)],
    )(embt)


def kernel(ids, emb_matrix):
    n_b, n_t = ids.shape
    vocab, dim = emb_matrix.shape
    ids_t = jnp.swapaxes(ids, 0, 1).astype(jnp.int32)
    table_lin = _linearize_table(emb_matrix).reshape(vocab, dim)
    out3 = _build(n_b, n_t, vocab)(ids_t, table_lin)
    return jnp.swapaxes(out3, 0, 1)
